# baseline (speedup 1.0000x reference)
import functools

import jax
import jax.numpy as jnp
from jax import lax
from jax.experimental import pallas as pl
from jax.experimental.pallas import tpu as pltpu
from jax.experimental.pallas import tpu_sc as plsc

D = 32
L = 50
CHUNK = 128
PK = CHUNK // 4
NC, NS = 2, 16
NW = NC * NS


GROUP = 8192
MAIN = (10 ** 6 // GROUP) * GROUP
TAILS = MAIN // 4


def _split_id(v):
    return (jnp.where(v < MAIN, (v >> 13) * 2048 + (v & 2047),
                      TAILS + ((v - MAIN) >> 2)),
            jnp.where(v < MAIN, (v >> 11) & 3, (v - MAIN) & 3))


def _prep_body(a_ref, b_ref, n_ref, sup_ref, q_ref, bt_ref, nsup_ref,
               nq_ref):
    at = a_ref[...].T
    sup_ref[...], q_ref[...] = _split_id(at)
    bt_ref[...] = b_ref[...].T
    nsup_ref[...], nq_ref[...] = _split_id(n_ref[...])


def _prep_idx(a, b, n2):
    n, m = a.shape
    out = jax.ShapeDtypeStruct((m, n), jnp.int32)
    outn = jax.ShapeDtypeStruct(n2.shape, jnp.int32)
    return pl.pallas_call(
        _prep_body,
        out_shape=(out, out, out, outn, outn),
    )(a, b, n2)


def _conv_body(a0_ref, a1_ref, a2_ref, a3_ref, w_ref, out_ref):
    w = w_ref[...]

    def dg(a_ref):
        return lax.dot_general(a_ref[...], w, (((0,), (0,)), ((), ())),
                               preferred_element_type=jnp.float32)

    out_ref[...] = jnp.concatenate([dg(a0_ref), dg(a1_ref), dg(a2_ref),
                                    dg(a3_ref)], axis=1)


def _convert_table(t_t, w):
    v = t_t.shape[1]
    nblk = MAIN // GROUP

    def spec(a):
        return pl.BlockSpec((D, GROUP // 4), lambda k, a=a: (0, 4 * k + a))

    return pl.pallas_call(
        _conv_body,
        grid=(nblk,),
        in_specs=[spec(0), spec(1), spec(2), spec(3),
                  pl.BlockSpec((D, D), lambda k: (0, 0))],
        out_specs=pl.BlockSpec((GROUP // 4, 128), lambda k: (k, 0)),
        out_shape=jax.ShapeDtypeStruct((v // 4, 128), jnp.float32),
    )(t_t, t_t, t_t, t_t, w)


def _packed_table(t, w):
    main = _convert_table(t.T, w)
    tail = (t[MAIN:] @ w).reshape((t.shape[0] - MAIN) // 4, 128)
    return main.at[TAILS:].set(tail)


_SC_PARAMS = dict(
    compiler_params=pltpu.CompilerParams(use_tc_tiling_on_sc=True,
                                         needs_layout_passes=False),
)
NBUF = 5


def _extract(rows, q16s, pk_v, lane16):
    chunk = rows.shape[0]
    for g in range(chunk // 16):
        rows16 = lane16 + g * 16
        q16 = q16s(g)
        src_lane0 = q16 * 32
        dst_row = rows16 >> 2
        dst_lane0 = (rows16 & 3) * 32
        for d in range(D):
            vals = plsc.load_gather(rows, [rows16, src_lane0 + d])
            plsc.store_scatter(pk_v, [dst_row, dst_lane0 + d], vals)


def _sc_gather_eur(r2e4, sup3, q3):
    nw, n_chunks, chunk = sup3.shape
    n_pk = nw * n_chunks * PK
    per_w_pk = n_chunks * PK

    mesh = plsc.VectorSubcoreMesh(core_axis_name="c", subcore_axis_name="s")

    @functools.partial(
        pl.kernel,
        mesh=mesh,
        out_type=jax.ShapeDtypeStruct((n_pk, 128), jnp.float32),
        scratch_types=(
            [pltpu.VMEM((n_chunks, chunk), jnp.int32),
             pltpu.VMEM((n_chunks, chunk), jnp.int32),
             pltpu.VMEM((PK, 128), jnp.float32)]
            + [pltpu.VMEM((chunk, 128), jnp.float32)] * NBUF
            + [pltpu.SemaphoreType.DMA] * NBUF
        ),
        **_SC_PARAMS,
    )
    def k(r2e_hbm, sup_hbm, q_hbm, eur_out, sup_v, q_v, pk_v, *bufsem):
        rows = bufsem[:NBUF]
        sems = bufsem[NBUF:]
        cid = lax.axis_index("c")
        sid = lax.axis_index("s")
        wid = sid * NC + cid
        base_pk = wid * per_w_pk
        lane16 = lax.broadcasted_iota(jnp.int32, (16,), 0)

        pltpu.sync_copy(sup_hbm.at[wid], sup_v)
        pltpu.sync_copy(q_hbm.at[wid], q_v)

        def start(j, b):
            pltpu.make_async_copy(r2e_hbm.at[sup_v.at[j]], rows[b],
                                  sems[b]).start()

        def wait(j, b):
            pltpu.make_async_copy(r2e_hbm.at[sup_v.at[j]], rows[b],
                                  sems[b]).wait()

        for b in range(NBUF):
            start(b, b)

        def body(i, carry):
            j0 = i * NBUF
            for b in range(NBUF):
                j = j0 + b
                wait(j, b)
                _extract(rows[b], lambda g: q_v[j, pl.ds(g * 16, 16)],
                         pk_v, lane16)
                pltpu.sync_copy(pk_v, eur_out.at[pl.ds(base_pk + j * PK, PK)])

                @pl.when(j + NBUF < n_chunks)
                def _():
                    start(j + NBUF, b)
            return carry

        lax.fori_loop(0, n_chunks // NBUF, body, 0)

    return k(r2e4, sup3, q3)


def _sc_gather_nodes(u2e4, nsup2, nq2):
    nw, chunk = nsup2.shape

    mesh = plsc.VectorSubcoreMesh(core_axis_name="c", subcore_axis_name="s")

    @functools.partial(
        pl.kernel,
        mesh=mesh,
        out_type=jax.ShapeDtypeStruct((nw * PK, 128), jnp.float32),
        scratch_types=[
            pltpu.VMEM((chunk,), jnp.int32),
            pltpu.VMEM((chunk,), jnp.int32),
            pltpu.VMEM((chunk, 128), jnp.float32),
            pltpu.VMEM((PK, 128), jnp.float32),
            pltpu.SemaphoreType.DMA,
        ],
        **_SC_PARAMS,
    )
    def k(u2e_hbm, nsup_hbm, nq_hbm, urep_out, nsup_v, nq_v, nrows_v,
          npk_v, nsem):
        cid = lax.axis_index("c")
        sid = lax.axis_index("s")
        wid = sid * NC + cid
        lane16 = lax.broadcasted_iota(jnp.int32, (16,), 0)

        pltpu.sync_copy(nsup_hbm.at[wid], nsup_v)
        pltpu.sync_copy(nq_hbm.at[wid], nq_v)
        pltpu.async_copy(u2e_hbm.at[nsup_v], nrows_v, nsem).wait()
        _extract(nrows_v, lambda g: nq_v[pl.ds(g * 16, 16)], npk_v, lane16)
        pltpu.sync_copy(npk_v, urep_out.at[pl.ds(wid * PK, PK)])

    return k(u2e4, nsup2, nq2)


def _tc_body(eur_ref, rat_ref, urep_ref, r2e_ref, w1b_ref, b1_ref,
             w2_ref, b2_ref, a1a_ref, a1b_ref, a2_ref, a2b_ref,
             a3_ref, a3b_ref, out_ref, ucon, m_run, d_run, acc):
    f32 = jnp.float32
    l = pl.program_id(0)
    mm = functools.partial(jnp.dot, preferred_element_type=f32)

    lane32 = lax.broadcasted_iota(jnp.int32, (1, 32), 1)
    g4_32 = (lax.broadcasted_iota(jnp.int32, (4, 32), 1) // 8
             == lax.broadcasted_iota(jnp.int32, (4, 32), 0)).astype(f32)
    g4_128 = (lax.broadcasted_iota(jnp.int32, (4, 128), 1) // 32
              == lax.broadcasted_iota(jnp.int32, (4, 128), 0)).astype(f32)

    @pl.when(l == 0)
    def _():
        ucon[...] = urep_ref[...] + a1b_ref[...]
        m_run[...] = jnp.full(m_run.shape, -1e30, f32)
        d_run[...] = jnp.zeros(d_run.shape, f32)
        acc[...] = jnp.zeros(acc.shape, f32)

    rproj = mm(r2e_ref[...], w1b_ref[...])
    ri = lax.broadcasted_iota(jnp.int32, (32, 128), 0)
    rj = lax.broadcasted_iota(jnp.int32, (32, 128), 1)
    rproj_bd = jnp.where(ri // 8 == rj // 32, jnp.tile(rproj, (4, 4)), 0.0)

    r4 = rat_ref[...].astype(f32)
    r_exp = mm(r4, g4_32)
    oh = (r_exp == (lane32 % 8).astype(f32)).astype(f32)

    x = eur_ref[...]
    h = jnp.maximum(x + mm(oh, rproj_bd) + b1_ref[...], 0.0)
    o = jnp.maximum(mm(h, w2_ref[...]) + b2_ref[...], 0.0)
    a1 = jnp.maximum(mm(o, a1a_ref[...]) + ucon[...], 0.0)
    a2 = jnp.maximum(mm(a1, a2_ref[...]) + a2b_ref[...], 0.0)
    s = mm(a2, a3_ref[...]) + a3b_ref[...]

    m_prev = m_run[...]
    m_new = jnp.maximum(m_prev, s)
    alpha = jnp.exp(m_prev - m_new)
    p = jnp.exp(s - m_new)
    m_run[...] = m_new
    d_new = d_run[...] * alpha + p
    d_run[...] = d_new
    acc_new = acc[...] * mm(alpha, g4_128) + mm(p, g4_128) * o
    acc[...] = acc_new

    @pl.when(l == L - 1)
    def _():
        out_ref[...] = acc_new / mm(d_new, g4_128)


def _tc_attention(eur_p, rat4, urep_p, r2e_pad, w1b, b1_bd, w2_bd,
                  b2_bd, a1a_bd, a1b_bd, a2_bd, a2b_bd, a3_bd, a3b_t):
    m = urep_p.shape[0]

    def c(shape):
        return pl.BlockSpec(shape, lambda l: (0, 0))

    specs = [
        pl.BlockSpec((m, 128), lambda l: (l, 0)),
        pl.BlockSpec((m, 4), lambda l: (l, 0)),
        c((m, 128)), c((8, D)), c((D, D)),
        c((1, 128)),
        c((128, 128)), c((1, 128)),
        c((128, 128)), c((1, 128)),
        c((128, 128)), c((1, 128)),
        c((128, 4)), c((1, 4)),
    ]
    return pl.pallas_call(
        _tc_body,
        grid=(L,),
        in_specs=specs,
        out_specs=pl.BlockSpec((m, 128), lambda l: (0, 0)),
        out_shape=jax.ShapeDtypeStruct((m, 128), jnp.float32),
        scratch_shapes=[
            pltpu.VMEM((m, 128), jnp.float32),
            pltpu.VMEM((m, 4), jnp.float32),
            pltpu.VMEM((m, 4), jnp.float32),
            pltpu.VMEM((m, 128), jnp.float32),
        ],
    )(eur_p, rat4, urep_p, r2e_pad, w1b, b1_bd, w2_bd, b2_bd,
      a1a_bd, a1b_bd, a2_bd, a2b_bd, a3_bd, a3b_t)


def kernel(nodes, ur_history_lists, rating_history_lists, u2e_w, r2e_w,
           rating2e_w, w_r1_w, w_r1_b, w_r2_w, w_r2_b, att1_w, att1_b,
           att2_w, att2_b, att3_w, att3_b):
    b_nodes = nodes.shape[0]
    n_rows = b_nodes * L
    per_w = n_rows // NW

    sup_t, q_t, rat_t, nsup2, nq2 = _prep_idx(
        ur_history_lists.astype(jnp.int32),
        rating_history_lists.astype(jnp.int32),
        nodes.astype(jnp.int32).reshape(NW, b_nodes // NW))
    sup3 = sup_t.reshape(NW, per_w // CHUNK, CHUNK)
    q3 = q_t.reshape(NW, per_w // CHUNK, CHUNK)

    r2e4 = _packed_table(r2e_w, w_r1_w[:D])
    eur_p = _sc_gather_eur(r2e4, sup3, q3)
    u2e4 = _packed_table(u2e_w, att1_w[D:])
    urep_p = _sc_gather_nodes(u2e4, nsup2, nq2)

    rat4 = rat_t.reshape(n_rows // 4, 4)
    r2e_pad = jnp.zeros((8, D), jnp.float32).at[:5].set(rating2e_w)

    eye4 = jnp.eye(4, dtype=jnp.float32)
    bd = lambda w: jnp.kron(eye4, w)
    t4 = lambda b: jnp.tile(b, 4).reshape(1, -1)

    out_p = _tc_attention(
        eur_p, rat4, urep_p, r2e_pad,
        w_r1_w[D:], t4(w_r1_b),
        bd(w_r2_w), t4(w_r2_b),
        bd(att1_w[:D]), t4(att1_b),
        bd(att2_w), t4(att2_b),
        bd(att3_w), t4(att3_b),
    )
    return out_p.reshape(b_nodes, D)


# speedup vs baseline: 1.0244x; 1.0244x over previous
import functools

import jax
import jax.numpy as jnp
from jax import lax
from jax.experimental import pallas as pl
from jax.experimental.pallas import tpu as pltpu
from jax.experimental.pallas import tpu_sc as plsc

D = 32
L = 50
CHUNK = 128
PK = CHUNK // 4
NC, NS = 2, 16
NW = NC * NS


GROUP = 16384
MAIN = (10 ** 6 // GROUP) * GROUP
TAILS = MAIN // 4


def _split_id(v):
    return (jnp.where(v < MAIN, (v >> 14) * 4096 + (v & 4095),
                      TAILS + ((v - MAIN) >> 2)),
            jnp.where(v < MAIN, (v >> 12) & 3, (v - MAIN) & 3))


def _prep_body(a_ref, b_ref, n_ref, sup_ref, q_ref, bt_ref, nsup_ref,
               nq_ref):
    at = a_ref[...].T
    sup_ref[...], q_ref[...] = _split_id(at)
    bt_ref[...] = b_ref[...].T
    nsup_ref[...], nq_ref[...] = _split_id(n_ref[...])


def _prep_idx(a, b, n2):
    n, m = a.shape
    out = jax.ShapeDtypeStruct((m, n), jnp.int32)
    outn = jax.ShapeDtypeStruct(n2.shape, jnp.int32)
    return pl.pallas_call(
        _prep_body,
        out_shape=(out, out, out, outn, outn),
    )(a, b, n2)


def _conv_body(a0_ref, a1_ref, a2_ref, a3_ref, w_ref, out_ref):
    w = w_ref[...]
    for a, ref in enumerate((a0_ref, a1_ref, a2_ref, a3_ref)):
        out_ref[:, a * D:(a + 1) * D] = lax.dot_general(
            ref[...], w, (((0,), (0,)), ((), ())),
            preferred_element_type=jnp.float32)


def _convert_table(t_t, w):
    v = t_t.shape[1]
    nblk = MAIN // GROUP

    def spec(a):
        return pl.BlockSpec((D, GROUP // 4), lambda k, a=a: (0, 4 * k + a))

    return pl.pallas_call(
        _conv_body,
        grid=(nblk,),
        in_specs=[spec(0), spec(1), spec(2), spec(3),
                  pl.BlockSpec((D, D), lambda k: (0, 0))],
        out_specs=pl.BlockSpec((GROUP // 4, 128), lambda k: (k, 0)),
        out_shape=jax.ShapeDtypeStruct((v // 4, 128), jnp.float32),
    )(t_t, t_t, t_t, t_t, w)


def _packed_table(t, w):
    main = _convert_table(t.T, w)
    tail = (t[MAIN:] @ w).reshape((t.shape[0] - MAIN) // 4, 128)
    return main.at[TAILS:].set(tail)


_SC_PARAMS = dict(
    compiler_params=pltpu.CompilerParams(use_tc_tiling_on_sc=True,
                                         needs_layout_passes=False),
)
NBUF = 5


def _extract(rows, q16s, pk_v, lane16):
    chunk = rows.shape[0]
    for g in range(chunk // 16):
        rows16 = lane16 + g * 16
        q16 = q16s(g)
        src_lane0 = q16 * 32
        dst_row = rows16 >> 2
        dst_lane0 = (rows16 & 3) * 32
        for d in range(D):
            vals = plsc.load_gather(rows, [rows16, src_lane0 + d])
            plsc.store_scatter(pk_v, [dst_row, dst_lane0 + d], vals)


def _sc_gather_eur(r2e4, sup3, q3):
    nw, n_chunks, chunk = sup3.shape
    n_pk = nw * n_chunks * PK
    per_w_pk = n_chunks * PK

    mesh = plsc.VectorSubcoreMesh(core_axis_name="c", subcore_axis_name="s")

    @functools.partial(
        pl.kernel,
        mesh=mesh,
        out_type=jax.ShapeDtypeStruct((n_pk, 128), jnp.float32),
        scratch_types=(
            [pltpu.VMEM((n_chunks, chunk), jnp.int32),
             pltpu.VMEM((n_chunks, chunk), jnp.int32),
             pltpu.VMEM((PK, 128), jnp.float32)]
            + [pltpu.VMEM((chunk, 128), jnp.float32)] * NBUF
            + [pltpu.SemaphoreType.DMA] * NBUF
        ),
        **_SC_PARAMS,
    )
    def k(r2e_hbm, sup_hbm, q_hbm, eur_out, sup_v, q_v, pk_v, *bufsem):
        rows = bufsem[:NBUF]
        sems = bufsem[NBUF:]
        cid = lax.axis_index("c")
        sid = lax.axis_index("s")
        wid = sid * NC + cid
        base_pk = wid * per_w_pk
        lane16 = lax.broadcasted_iota(jnp.int32, (16,), 0)

        pltpu.sync_copy(sup_hbm.at[wid], sup_v)
        pltpu.sync_copy(q_hbm.at[wid], q_v)

        def start(j, b):
            pltpu.make_async_copy(r2e_hbm.at[sup_v.at[j]], rows[b],
                                  sems[b]).start()

        def wait(j, b):
            pltpu.make_async_copy(r2e_hbm.at[sup_v.at[j]], rows[b],
                                  sems[b]).wait()

        for b in range(NBUF):
            start(b, b)

        def body(i, carry):
            j0 = i * NBUF
            for b in range(NBUF):
                j = j0 + b
                wait(j, b)
                _extract(rows[b], lambda g: q_v[j, pl.ds(g * 16, 16)],
                         pk_v, lane16)
                pltpu.sync_copy(pk_v, eur_out.at[pl.ds(base_pk + j * PK, PK)])

                @pl.when(j + NBUF < n_chunks)
                def _():
                    start(j + NBUF, b)
            return carry

        lax.fori_loop(0, n_chunks // NBUF, body, 0)

    return k(r2e4, sup3, q3)


def _sc_gather_nodes(u2e4, nsup2, nq2):
    nw, chunk = nsup2.shape

    mesh = plsc.VectorSubcoreMesh(core_axis_name="c", subcore_axis_name="s")

    @functools.partial(
        pl.kernel,
        mesh=mesh,
        out_type=jax.ShapeDtypeStruct((nw * PK, 128), jnp.float32),
        scratch_types=[
            pltpu.VMEM((chunk,), jnp.int32),
            pltpu.VMEM((chunk,), jnp.int32),
            pltpu.VMEM((chunk, 128), jnp.float32),
            pltpu.VMEM((PK, 128), jnp.float32),
            pltpu.SemaphoreType.DMA,
        ],
        **_SC_PARAMS,
    )
    def k(u2e_hbm, nsup_hbm, nq_hbm, urep_out, nsup_v, nq_v, nrows_v,
          npk_v, nsem):
        cid = lax.axis_index("c")
        sid = lax.axis_index("s")
        wid = sid * NC + cid
        lane16 = lax.broadcasted_iota(jnp.int32, (16,), 0)

        pltpu.sync_copy(nsup_hbm.at[wid], nsup_v)
        pltpu.sync_copy(nq_hbm.at[wid], nq_v)
        pltpu.async_copy(u2e_hbm.at[nsup_v], nrows_v, nsem).wait()
        _extract(nrows_v, lambda g: nq_v[pl.ds(g * 16, 16)], npk_v, lane16)
        pltpu.sync_copy(npk_v, urep_out.at[pl.ds(wid * PK, PK)])

    return k(u2e4, nsup2, nq2)


def _tc_body(eur_ref, rat_ref, urep_ref, r2e_ref, w1b_ref, b1_ref,
             w2_ref, b2_ref, a1a_ref, a1b_ref, a2_ref, a2b_ref,
             a3_ref, a3b_ref, out_ref, ucon, m_run, d_run, acc):
    f32 = jnp.float32
    l = pl.program_id(0)
    mm = functools.partial(jnp.dot, preferred_element_type=f32)

    lane32 = lax.broadcasted_iota(jnp.int32, (1, 32), 1)
    g4_32 = (lax.broadcasted_iota(jnp.int32, (4, 32), 1) // 8
             == lax.broadcasted_iota(jnp.int32, (4, 32), 0)).astype(f32)
    g4_128 = (lax.broadcasted_iota(jnp.int32, (4, 128), 1) // 32
              == lax.broadcasted_iota(jnp.int32, (4, 128), 0)).astype(f32)

    @pl.when(l == 0)
    def _():
        ucon[...] = urep_ref[...] + a1b_ref[...]
        m_run[...] = jnp.full(m_run.shape, -1e30, f32)
        d_run[...] = jnp.zeros(d_run.shape, f32)
        acc[...] = jnp.zeros(acc.shape, f32)

    rproj = mm(r2e_ref[...], w1b_ref[...])
    ri = lax.broadcasted_iota(jnp.int32, (32, 128), 0)
    rj = lax.broadcasted_iota(jnp.int32, (32, 128), 1)
    rproj_bd = jnp.where(ri // 8 == rj // 32, jnp.tile(rproj, (4, 4)), 0.0)

    r4 = rat_ref[...].astype(f32)
    r_exp = mm(r4, g4_32)
    oh = (r_exp == (lane32 % 8).astype(f32)).astype(f32)

    x = eur_ref[...]
    h = jnp.maximum(x + mm(oh, rproj_bd) + b1_ref[...], 0.0)
    o = jnp.maximum(mm(h, w2_ref[...]) + b2_ref[...], 0.0)
    a1 = jnp.maximum(mm(o, a1a_ref[...]) + ucon[...], 0.0)
    a2 = jnp.maximum(mm(a1, a2_ref[...]) + a2b_ref[...], 0.0)
    s = mm(a2, a3_ref[...]) + a3b_ref[...]

    m_prev = m_run[...]
    m_new = jnp.maximum(m_prev, s)
    alpha = jnp.exp(m_prev - m_new)
    p = jnp.exp(s - m_new)
    m_run[...] = m_new
    d_new = d_run[...] * alpha + p
    d_run[...] = d_new
    acc_new = acc[...] * mm(alpha, g4_128) + mm(p, g4_128) * o
    acc[...] = acc_new

    @pl.when(l == L - 1)
    def _():
        out_ref[...] = acc_new / mm(d_new, g4_128)


def _tc_attention(eur_p, rat4, urep_p, r2e_pad, w1b, b1_bd, w2_bd,
                  b2_bd, a1a_bd, a1b_bd, a2_bd, a2b_bd, a3_bd, a3b_t):
    m = urep_p.shape[0]

    def c(shape):
        return pl.BlockSpec(shape, lambda l: (0, 0))

    specs = [
        pl.BlockSpec((m, 128), lambda l: (l, 0)),
        pl.BlockSpec((m, 4), lambda l: (l, 0)),
        c((m, 128)), c((8, D)), c((D, D)),
        c((1, 128)),
        c((128, 128)), c((1, 128)),
        c((128, 128)), c((1, 128)),
        c((128, 128)), c((1, 128)),
        c((128, 4)), c((1, 4)),
    ]
    return pl.pallas_call(
        _tc_body,
        grid=(L,),
        in_specs=specs,
        out_specs=pl.BlockSpec((m, 128), lambda l: (0, 0)),
        out_shape=jax.ShapeDtypeStruct((m, 128), jnp.float32),
        scratch_shapes=[
            pltpu.VMEM((m, 128), jnp.float32),
            pltpu.VMEM((m, 4), jnp.float32),
            pltpu.VMEM((m, 4), jnp.float32),
            pltpu.VMEM((m, 128), jnp.float32),
        ],
    )(eur_p, rat4, urep_p, r2e_pad, w1b, b1_bd, w2_bd, b2_bd,
      a1a_bd, a1b_bd, a2_bd, a2b_bd, a3_bd, a3b_t)


def kernel(nodes, ur_history_lists, rating_history_lists, u2e_w, r2e_w,
           rating2e_w, w_r1_w, w_r1_b, w_r2_w, w_r2_b, att1_w, att1_b,
           att2_w, att2_b, att3_w, att3_b):
    b_nodes = nodes.shape[0]
    n_rows = b_nodes * L
    per_w = n_rows // NW

    sup_t, q_t, rat_t, nsup2, nq2 = _prep_idx(
        ur_history_lists.astype(jnp.int32),
        rating_history_lists.astype(jnp.int32),
        nodes.astype(jnp.int32).reshape(NW, b_nodes // NW))
    sup3 = sup_t.reshape(NW, per_w // CHUNK, CHUNK)
    q3 = q_t.reshape(NW, per_w // CHUNK, CHUNK)

    r2e4 = _packed_table(r2e_w, w_r1_w[:D])
    eur_p = _sc_gather_eur(r2e4, sup3, q3)
    u2e4 = _packed_table(u2e_w, att1_w[D:])
    urep_p = _sc_gather_nodes(u2e4, nsup2, nq2)

    rat4 = rat_t.reshape(n_rows // 4, 4)
    r2e_pad = jnp.zeros((8, D), jnp.float32).at[:5].set(rating2e_w)

    eye4 = jnp.eye(4, dtype=jnp.float32)
    bd = lambda w: jnp.kron(eye4, w)
    t4 = lambda b: jnp.tile(b, 4).reshape(1, -1)

    out_p = _tc_attention(
        eur_p, rat4, urep_p, r2e_pad,
        w_r1_w[D:], t4(w_r1_b),
        bd(w_r2_w), t4(w_r2_b),
        bd(att1_w[:D]), t4(att1_b),
        bd(att2_w), t4(att2_b),
        bd(att3_w), t4(att3_b),
    )
    return out_p.reshape(b_nodes, D)
